# baseline (speedup 1.0000x reference)
import functools
import math

import jax
import jax.numpy as jnp
from jax import lax
from jax.experimental import pallas as pl
from jax.experimental.pallas import tpu as pltpu
from jax.experimental.pallas import tpu_sc as plsc

D_MODEL = 64
SEQ = 200
BATCH = 1024
ROWS = BATCH * SEQ
NVOCAB = 1000000
NPAIR = NVOCAB // 2
SCALE = math.sqrt(D_MODEL)

BLK_P = 72
BLK_R = 2 * BLK_P
NBLK = 6944
TAIL_R0 = NBLK * BLK_R
NBUF0 = 3

CHUNK_B = 128
NBUF_B = 2


def _k0_body(nw, table_hbm, pairs_hbm, vbuf, obuf, vtail, otail, *sems):
    gsem = sems[:NBUF0]
    ssem = sems[NBUF0:]
    nc = nw // 16
    wid = lax.axis_index("s") * nc + lax.axis_index("c")
    nblk_w = NBLK // nw
    start = wid * nblk_w

    @pl.when(wid == nw - 1)
    def _():
        pltpu.sync_copy(table_hbm.at[pl.ds(TAIL_R0, 64)], vtail)
        for p in range(32):
            for h in range(2):
                for d0 in range(0, D_MODEL, 16):
                    otail[p, pl.ds(h * D_MODEL + d0, 16)] = (
                        vtail[2 * p + h, pl.ds(d0, 16)])
        pltpu.sync_copy(otail, pairs_hbm.at[pl.ds(TAIL_R0 // 2, 32)])

    def gather(bg, b):
        off = pl.multiple_of(bg * BLK_R, 8)
        return pltpu.make_async_copy(
            table_hbm.at[pl.ds(off, BLK_R)], vbuf.at[b], gsem[b])

    def scat(bg, b):
        off = pl.multiple_of(bg * BLK_P, 8)
        return pltpu.make_async_copy(
            obuf.at[b], pairs_hbm.at[pl.ds(off, BLK_P)], ssem[b])

    ngrp = (nblk_w + NBUF0 - 1) // NBUF0

    for b in range(NBUF0):
        gather(start + b, b).start()

    def grp(g, _):
        for b in range(NBUF0):
            i = g * NBUF0 + b
            bg = start + i

            @pl.when(i < nblk_w)
            def _():
                gather(bg, b).wait()

                @pl.when(g > 0)
                def _():
                    scat(bg - NBUF0, b).wait()

                @plsc.parallel_loop(0, BLK_P)
                def _(p):
                    for h in range(2):
                        for d0 in range(0, D_MODEL, 16):
                            obuf[b, p, pl.ds(h * D_MODEL + d0, 16)] = (
                                vbuf[b, 2 * p + h, pl.ds(d0, 16)])

                scat(bg, b).start()

                @pl.when(i + NBUF0 < nblk_w)
                def _():
                    gather(bg + NBUF0, b).start()
        return 0

    lax.fori_loop(0, ngrp, grp, 0)

    for b in range(NBUF0):
        im1 = nblk_w - 1
        ib = im1 - lax.rem(im1 - b, NBUF0)
        scat(start + ib, b).wait()


def _b_body(nw, nchunk, idxp_hbm, idxf_hbm, pairs_hbm, pe_hbm, out_hbm,
            idxp_v, idxf_v, pe_v, rows_v, obuf_v, *sems):
    gsem = sems[:NBUF_B]
    ssem = sems[NBUF_B:]
    nc = nw // 16
    wid = lax.axis_index("s") * nc + lax.axis_index("c")
    base = wid * nchunk

    pltpu.sync_copy(idxp_hbm.at[wid], idxp_v)
    pltpu.sync_copy(idxf_hbm.at[wid], idxf_v)
    pltpu.sync_copy(pe_hbm, pe_v)

    def gather(c, b):
        return pltpu.make_async_copy(
            pairs_hbm.at[idxp_v.at[c]], rows_v.at[b], gsem[b])

    def scat(c, b):
        return pltpu.make_async_copy(
            obuf_v.at[b],
            out_hbm.at[pl.ds((base + c) * CHUNK_B, CHUNK_B)], ssem[b])

    for b in range(NBUF_B):
        gather(b, b).start()

    ngroup = nchunk // NBUF_B

    def grp(g, _):
        for b in range(NBUF_B):
            c = g * NBUF_B + b
            gather(c, b).wait()

            @pl.when(g > 0)
            def _():
                scat(c - NBUF_B, b).wait()

            sbase = lax.rem((base + c) * CHUNK_B, SEQ)

            @plsc.parallel_loop(0, CHUNK_B // 16)
            def _(grp16):
                vsub = lax.rem(idxf_v[c, pl.ds(grp16 * 16, 16)], 2)
                for k in range(16):
                    sub = vsub[k]
                    r = grp16 * 16 + k
                    s = lax.rem(sbase + r, SEQ)
                    w0 = s * D_MODEL
                    r0 = lax.shift_right_logical(w0, 7)
                    o0 = lax.bitwise_and(w0, 127)
                    for d in range(D_MODEL // 16):
                        obuf_v[b, r, pl.ds(d * 16, 16)] = (
                            rows_v[b, r, pl.ds(sub * D_MODEL + d * 16, 16)]
                            * SCALE
                            + pe_v[r0, pl.ds(o0 + d * 16, 16)])

            scat(c, b).start()

            @pl.when(g < ngroup - 1)
            def _():
                gather(c + NBUF_B, b).start()
        return 0

    lax.fori_loop(0, ngroup, grp, 0)

    for b in range(NBUF_B):
        scat(nchunk - NBUF_B + b, b).wait()


def kernel(x, table, pe):
    info = plsc.get_sparse_core_info()
    nw = info.num_cores * info.num_subcores
    nchunk = ROWS // (nw * CHUNK_B)

    xi = x.astype(jnp.int32).reshape(ROWS)
    idxp = (xi // 2).reshape(nw, nchunk, CHUNK_B)
    idxf = xi.reshape(nw, nchunk, CHUNK_B)
    pe128 = pe[:SEQ, 0, :].reshape(SEQ * D_MODEL // 128, 128)

    mesh = plsc.VectorSubcoreMesh(core_axis_name="c", subcore_axis_name="s")
    k0 = pl.kernel(
        functools.partial(_k0_body, nw),
        mesh=mesh,
        compiler_params=pltpu.CompilerParams(use_tc_tiling_on_sc=True),
        out_type=jax.ShapeDtypeStruct((NPAIR, 128), jnp.float32),
        scratch_types=[
            pltpu.VMEM((NBUF0, BLK_R, D_MODEL), jnp.float32),
            pltpu.VMEM((NBUF0, BLK_P, 128), jnp.float32),
            pltpu.VMEM((64, D_MODEL), jnp.float32),
            pltpu.VMEM((32, 128), jnp.float32),
        ] + [pltpu.SemaphoreType.DMA] * (2 * NBUF0),
    )
    pairs = k0(table)

    fb = pl.kernel(
        functools.partial(_b_body, nw, nchunk),
        mesh=mesh,
        compiler_params=pltpu.CompilerParams(use_tc_tiling_on_sc=True),
        out_type=jax.ShapeDtypeStruct((ROWS, 128), jnp.float32),
        scratch_types=[
            pltpu.VMEM((nchunk, CHUNK_B), jnp.int32),
            pltpu.VMEM((nchunk, CHUNK_B), jnp.int32),
            pltpu.VMEM((SEQ * D_MODEL // 128, 128), jnp.float32),
            pltpu.VMEM((NBUF_B, CHUNK_B, 128), jnp.float32),
            pltpu.VMEM((NBUF_B, CHUNK_B, 128), jnp.float32),
        ] + [pltpu.SemaphoreType.DMA] * (2 * NBUF_B),
    )
    out128 = fb(idxp, idxf, pairs, pe128)
    return out128[:, :D_MODEL].reshape(BATCH, SEQ, D_MODEL)


# speedup vs baseline: 1.1644x; 1.1644x over previous
import functools
import math

import jax
import jax.numpy as jnp
from jax import lax
from jax.experimental import pallas as pl
from jax.experimental.pallas import tpu as pltpu
from jax.experimental.pallas import tpu_sc as plsc

D_MODEL = 64
SEQ = 200
BATCH = 1024
ROWS = BATCH * SEQ
NVOCAB = 1000000
NPAIR = NVOCAB // 2
SCALE = math.sqrt(D_MODEL)

BLK_P = 72
BLK_R = 2 * BLK_P
NBLK = 6944
TAIL_R0 = NBLK * BLK_R
NBUF0 = 3

CHUNK_B = 128
NBUF_B = 3


def _k0_body(nw, table_hbm, pairs_hbm, vbuf, obuf, vtail, otail, *sems):
    gsem = sems[:NBUF0]
    ssem = sems[NBUF0:]
    nc = nw // 16
    wid = lax.axis_index("s") * nc + lax.axis_index("c")
    nblk_w = NBLK // nw
    start = wid * nblk_w

    @pl.when(wid == nw - 1)
    def _():
        pltpu.sync_copy(table_hbm.at[pl.ds(TAIL_R0 // 8, 8)], vtail)
        for p in range(32):
            for h in range(2):
                rr = 2 * p + h
                for d0 in range(0, D_MODEL, 16):
                    otail[p, pl.ds(h * D_MODEL + d0, 16)] = (
                        vtail[rr // 8, rr % 8, pl.ds(d0, 16)])
        pltpu.sync_copy(otail, pairs_hbm.at[pl.ds(TAIL_R0 // 2, 32)])

    def gather(bg, b):
        off = pl.multiple_of(bg * (BLK_R // 8), 2)
        return pltpu.make_async_copy(
            table_hbm.at[pl.ds(off, BLK_R // 8)], vbuf.at[b], gsem[b])

    def scat(bg, b):
        off = pl.multiple_of(bg * BLK_P, 8)
        return pltpu.make_async_copy(
            obuf.at[b], pairs_hbm.at[pl.ds(off, BLK_P)], ssem[b])

    ngrp = (nblk_w + NBUF0 - 1) // NBUF0

    for b in range(NBUF0):
        gather(start + b, b).start()

    def grp(g, _):
        for b in range(NBUF0):
            i = g * NBUF0 + b
            bg = start + i

            @pl.when(i < nblk_w)
            def _():
                gather(bg, b).wait()

                @pl.when(g > 0)
                def _():
                    scat(bg - NBUF0, b).wait()

                @plsc.parallel_loop(0, BLK_R // 8)
                def _(t8):
                    for row in range(8):
                        for d0 in range(0, D_MODEL, 16):
                            obuf[b, t8 * 4 + row // 2,
                                 pl.ds((row % 2) * D_MODEL + d0, 16)] = (
                                vbuf[b, t8, row, pl.ds(d0, 16)])

                scat(bg, b).start()

                @pl.when(i + NBUF0 < nblk_w)
                def _():
                    gather(bg + NBUF0, b).start()
        return 0

    lax.fori_loop(0, ngrp, grp, 0)

    for b in range(NBUF0):
        im1 = nblk_w - 1
        ib = im1 - lax.rem(im1 - b, NBUF0)
        scat(start + ib, b).wait()


def _b_body(nw, nchunk, idxp_hbm, idxf_hbm, pairs_hbm, pe_hbm, out_hbm,
            idxp_v, idxf_v, pe_v, rows_v, obuf_v, *sems):
    gsem = sems[:NBUF_B]
    ssem = sems[NBUF_B:]
    nc = nw // 16
    wid = lax.axis_index("s") * nc + lax.axis_index("c")
    base = wid * nchunk

    pltpu.sync_copy(idxp_hbm.at[wid], idxp_v)
    pltpu.sync_copy(idxf_hbm.at[wid], idxf_v)
    pltpu.sync_copy(pe_hbm, pe_v)

    def gather(c, b):
        return pltpu.make_async_copy(
            pairs_hbm.at[idxp_v.at[c]], rows_v.at[b], gsem[b])

    def scat(c, b):
        return pltpu.make_async_copy(
            obuf_v.at[b],
            out_hbm.at[pl.ds((base + c) * CHUNK_B, CHUNK_B)], ssem[b])

    for b in range(NBUF_B):
        gather(b, b).start()

    ngroup = nchunk // NBUF_B

    def grp(g, _):
        for b in range(NBUF_B):
            c = g * NBUF_B + b
            gather(c, b).wait()

            @pl.when(g > 0)
            def _():
                scat(c - NBUF_B, b).wait()

            sbase = lax.rem((base + c) * CHUNK_B, SEQ)

            @plsc.parallel_loop(0, CHUNK_B // 16)
            def _(grp16):
                vsub = lax.rem(idxf_v[c, pl.ds(grp16 * 16, 16)], 2)
                for k in range(16):
                    sub = vsub[k]
                    r = grp16 * 16 + k
                    s = lax.rem(sbase + r, SEQ)
                    w0 = s * D_MODEL
                    r0 = lax.shift_right_logical(w0, 7)
                    o0 = lax.bitwise_and(w0, 127)
                    for d in range(D_MODEL // 16):
                        obuf_v[b, r, pl.ds(d * 16, 16)] = (
                            rows_v[b, r, pl.ds(sub * D_MODEL + d * 16, 16)]
                            * SCALE
                            + pe_v[r0, pl.ds(o0 + d * 16, 16)])

            scat(c, b).start()

            @pl.when(g < ngroup - 1)
            def _():
                gather(c + NBUF_B, b).start()
        return 0

    lax.fori_loop(0, ngroup, grp, 0)

    for b in range(NBUF_B):
        scat(nchunk - NBUF_B + b, b).wait()


def kernel(x, table, pe):
    info = plsc.get_sparse_core_info()
    nw = info.num_cores * info.num_subcores
    nchunk = ROWS // (nw * CHUNK_B)

    xi = x.astype(jnp.int32).reshape(ROWS)
    idxp = (xi // 2).reshape(nw, nchunk, CHUNK_B)
    idxf = xi.reshape(nw, nchunk, CHUNK_B)
    pe128 = pe[:SEQ, 0, :].reshape(SEQ * D_MODEL // 128, 128)

    mesh = plsc.VectorSubcoreMesh(core_axis_name="c", subcore_axis_name="s")
    k0 = pl.kernel(
        functools.partial(_k0_body, nw),
        mesh=mesh,
        compiler_params=pltpu.CompilerParams(use_tc_tiling_on_sc=True),
        out_type=jax.ShapeDtypeStruct((NPAIR, 128), jnp.float32),
        scratch_types=[
            pltpu.VMEM((NBUF0, BLK_R // 8, 8, D_MODEL), jnp.float32),
            pltpu.VMEM((NBUF0, BLK_P, 128), jnp.float32),
            pltpu.VMEM((8, 8, D_MODEL), jnp.float32),
            pltpu.VMEM((32, 128), jnp.float32),
        ] + [pltpu.SemaphoreType.DMA] * (2 * NBUF0),
    )
    table3 = table.reshape(NVOCAB // 8, 8, D_MODEL)
    pairs = k0(table3)

    fb = pl.kernel(
        functools.partial(_b_body, nw, nchunk),
        mesh=mesh,
        compiler_params=pltpu.CompilerParams(use_tc_tiling_on_sc=True),
        out_type=jax.ShapeDtypeStruct((ROWS, 128), jnp.float32),
        scratch_types=[
            pltpu.VMEM((nchunk, CHUNK_B), jnp.int32),
            pltpu.VMEM((nchunk, CHUNK_B), jnp.int32),
            pltpu.VMEM((SEQ * D_MODEL // 128, 128), jnp.float32),
            pltpu.VMEM((NBUF_B, CHUNK_B, 128), jnp.float32),
            pltpu.VMEM((NBUF_B, CHUNK_B, 128), jnp.float32),
        ] + [pltpu.SemaphoreType.DMA] * (2 * NBUF_B),
    )
    out128 = fb(idxp, idxf, pairs, pe128)
    return out128[:, :D_MODEL].reshape(BATCH, SEQ, D_MODEL)


# speedup vs baseline: 1.1726x; 1.0071x over previous
import functools
import math

import jax
import jax.numpy as jnp
from jax import lax
from jax.experimental import pallas as pl
from jax.experimental.pallas import tpu as pltpu
from jax.experimental.pallas import tpu_sc as plsc

D_MODEL = 64
SEQ = 200
BATCH = 1024
ROWS = BATCH * SEQ
NVOCAB = 1000000
NPAIR = NVOCAB // 2
SCALE = math.sqrt(D_MODEL)

BLK_P = 72
BLK_R = 2 * BLK_P
NBLK = 6944
TAIL_R0 = NBLK * BLK_R
NBUF0 = 3

CHUNK_B = 128
NBUF_B = 2


def _k0_body(nw, table_hbm, pairs_hbm, vbuf, obuf, vtail, otail, *sems):
    gsem = sems[:NBUF0]
    ssem = sems[NBUF0:]
    nc = nw // 16
    wid = lax.axis_index("s") * nc + lax.axis_index("c")
    nblk_w = NBLK // nw
    start = wid * nblk_w

    @pl.when(wid == nw - 1)
    def _():
        pltpu.sync_copy(table_hbm.at[pl.ds(TAIL_R0 // 8, 8)], vtail)
        for p in range(32):
            for h in range(2):
                rr = 2 * p + h
                for d0 in range(0, D_MODEL, 16):
                    otail[p, pl.ds(h * D_MODEL + d0, 16)] = (
                        vtail[rr // 8, rr % 8, pl.ds(d0, 16)])
        pltpu.sync_copy(otail, pairs_hbm.at[pl.ds(TAIL_R0 // 2, 32)])

    def gather(bg, b):
        off = pl.multiple_of(bg * (BLK_R // 8), 2)
        return pltpu.make_async_copy(
            table_hbm.at[pl.ds(off, BLK_R // 8)], vbuf.at[b], gsem[b])

    def scat(bg, b):
        off = pl.multiple_of(bg * BLK_P, 8)
        return pltpu.make_async_copy(
            obuf.at[b], pairs_hbm.at[pl.ds(off, BLK_P)], ssem[b])

    ngrp = (nblk_w + NBUF0 - 1) // NBUF0

    for b in range(NBUF0):
        gather(start + b, b).start()

    def grp(g, _):
        for b in range(NBUF0):
            i = g * NBUF0 + b
            bg = start + i

            @pl.when(i < nblk_w)
            def _():
                gather(bg, b).wait()

                @pl.when(g > 0)
                def _():
                    scat(bg - NBUF0, b).wait()

                @plsc.parallel_loop(0, BLK_R // 8)
                def _(t8):
                    for row in range(8):
                        for d0 in range(0, D_MODEL, 16):
                            obuf[b, t8 * 4 + row // 2,
                                 pl.ds((row % 2) * D_MODEL + d0, 16)] = (
                                vbuf[b, t8, row, pl.ds(d0, 16)])

                scat(bg, b).start()

                @pl.when(i + NBUF0 < nblk_w)
                def _():
                    gather(bg + NBUF0, b).start()
        return 0

    lax.fori_loop(0, ngrp, grp, 0)

    for b in range(NBUF0):
        im1 = nblk_w - 1
        ib = im1 - lax.rem(im1 - b, NBUF0)
        scat(start + ib, b).wait()


def _b_body(nw, nchunk, idxp_hbm, idxf_hbm, pairs_hbm, pe_hbm, out_hbm,
            idxp_v, idxf_v, pe_v, rows_v, obuf_v, *sems):
    gsem = sems[:NBUF_B]
    ssem = sems[NBUF_B:]
    nc = nw // 16
    wid = lax.axis_index("s") * nc + lax.axis_index("c")
    base = wid * nchunk

    pltpu.sync_copy(idxp_hbm.at[wid], idxp_v)
    pltpu.sync_copy(idxf_hbm.at[wid], idxf_v)
    pltpu.sync_copy(pe_hbm, pe_v)

    def gather(c, b):
        return pltpu.make_async_copy(
            pairs_hbm.at[idxp_v.at[c]], rows_v.at[b], gsem[b])

    def scat(c, b):
        return pltpu.make_async_copy(
            obuf_v.at[b],
            out_hbm.at[pl.ds((base + c) * CHUNK_B, CHUNK_B)], ssem[b])

    for b in range(NBUF_B):
        gather(b, b).start()

    ngroup = nchunk // NBUF_B

    def grp(g, _):
        for b in range(NBUF_B):
            c = g * NBUF_B + b
            gather(c, b).wait()

            @pl.when(g > 0)
            def _():
                scat(c - NBUF_B, b).wait()

            sbase = lax.rem((base + c) * CHUNK_B, SEQ)

            @plsc.parallel_loop(0, CHUNK_B // 16)
            def _(grp16):
                vsub = lax.rem(idxf_v[c, pl.ds(grp16 * 16, 16)], 2)
                for k in range(16):
                    sub = vsub[k]
                    r = grp16 * 16 + k
                    s = lax.rem(sbase + r, SEQ)
                    w0 = s * D_MODEL
                    r0 = lax.shift_right_logical(w0, 7)
                    o0 = lax.bitwise_and(w0, 127)
                    for d in range(D_MODEL // 16):
                        obuf_v[b, r, pl.ds(d * 16, 16)] = (
                            rows_v[b, r, pl.ds(sub * D_MODEL + d * 16, 16)]
                            * SCALE
                            + pe_v[r0, pl.ds(o0 + d * 16, 16)])

            scat(c, b).start()

            @pl.when(g < ngroup - 1)
            def _():
                gather(c + NBUF_B, b).start()
        return 0

    lax.fori_loop(0, ngroup, grp, 0)

    for b in range(NBUF_B):
        scat(nchunk - NBUF_B + b, b).wait()


def kernel(x, table, pe):
    info = plsc.get_sparse_core_info()
    nw = info.num_cores * info.num_subcores
    nchunk = ROWS // (nw * CHUNK_B)

    xi = x.astype(jnp.int32).reshape(ROWS)
    idxp = (xi // 2).reshape(nw, nchunk, CHUNK_B)
    idxf = xi.reshape(nw, nchunk, CHUNK_B)
    pe128 = pe[:SEQ, 0, :].reshape(SEQ * D_MODEL // 128, 128)

    mesh = plsc.VectorSubcoreMesh(core_axis_name="c", subcore_axis_name="s")
    k0 = pl.kernel(
        functools.partial(_k0_body, nw),
        mesh=mesh,
        compiler_params=pltpu.CompilerParams(use_tc_tiling_on_sc=True),
        out_type=jax.ShapeDtypeStruct((NPAIR, 128), jnp.float32),
        scratch_types=[
            pltpu.VMEM((NBUF0, BLK_R // 8, 8, D_MODEL), jnp.float32),
            pltpu.VMEM((NBUF0, BLK_P, 128), jnp.float32),
            pltpu.VMEM((8, 8, D_MODEL), jnp.float32),
            pltpu.VMEM((32, 128), jnp.float32),
        ] + [pltpu.SemaphoreType.DMA] * (2 * NBUF0),
    )
    table3 = table.reshape(NVOCAB // 8, 8, D_MODEL)
    pairs = k0(table3)

    fb = pl.kernel(
        functools.partial(_b_body, nw, nchunk),
        mesh=mesh,
        compiler_params=pltpu.CompilerParams(use_tc_tiling_on_sc=True),
        out_type=jax.ShapeDtypeStruct((ROWS, 128), jnp.float32),
        scratch_types=[
            pltpu.VMEM((nchunk, CHUNK_B), jnp.int32),
            pltpu.VMEM((nchunk, CHUNK_B), jnp.int32),
            pltpu.VMEM((SEQ * D_MODEL // 128, 128), jnp.float32),
            pltpu.VMEM((NBUF_B, CHUNK_B, 128), jnp.float32),
            pltpu.VMEM((NBUF_B, CHUNK_B, 128), jnp.float32),
        ] + [pltpu.SemaphoreType.DMA] * (2 * NBUF_B),
    )
    out128 = fb(idxp, idxf, pairs, pe128)
    return out128[:, :D_MODEL].reshape(BATCH, SEQ, D_MODEL)
